# baseline (speedup 1.0000x reference)
import functools

import jax
import jax.numpy as jnp
import numpy as np
from jax import lax
from jax.experimental import pallas as pl
from jax.experimental.pallas import tpu as pltpu
from jax.experimental.pallas import tpu_sc as plsc

D = 640
N = 100000
Q = 1024
KTOP = 3
NLAB = 12
NLAYER = 3
RATIO = 0.2

BN = 2048
NBLK = (N + BN - 1) // BN
RQ = 128
NEG = float("-inf")


def _bdot(a, b):
    return jnp.dot(a, b, preferred_element_type=jnp.float32)


def _round_bf16(a):
    return a.astype(jnp.bfloat16).astype(jnp.float32)


def _search_body(q_ref, db_ref, dbt_ref, ts_ref, ti_ref, qn_ref, f_ref, c_ref):
    i = pl.program_id(0)

    @pl.when(i == 0)
    def _init():
        q = q_ref[...]
        nrm = jnp.sqrt(jnp.sum(q * q, axis=1, keepdims=True))
        qn_ref[...] = q / nrm
        f_ref[...] = jnp.full((KTOP, Q, 128), NEG, jnp.float32)
        c_ref[...] = jnp.zeros((KTOP, Q, 128), jnp.int32)

    db = db_ref[...]
    dbt_ref[...] = db.T
    qn = qn_ref[...]
    lane = lax.broadcasted_iota(jnp.int32, (Q, 128), 1)
    base = i * BN
    f1 = f_ref[0]
    f2 = f_ref[1]
    f3 = f_ref[2]
    c1 = c_ref[0]
    c2 = c_ref[1]
    c3 = c_ref[2]
    for g in range(BN // 128):
        v = jnp.dot(qn, db[:, g * 128:(g + 1) * 128],
                    preferred_element_type=jnp.float32)
        gb = base + g * 128
        v = jnp.where(lane < (N - gb), v, NEG)
        cc = lane + gb
        gt1 = v > f1
        gt2 = v > f2
        gt3 = v > f3
        nf3 = jnp.where(gt2, f2, jnp.where(gt3, v, f3))
        nc3 = jnp.where(gt2, c2, jnp.where(gt3, cc, c3))
        nf2 = jnp.where(gt1, f1, jnp.where(gt2, v, f2))
        nc2 = jnp.where(gt1, c1, jnp.where(gt2, cc, c2))
        nf1 = jnp.where(gt1, v, f1)
        nc1 = jnp.where(gt1, cc, c1)
        f1, f2, f3, c1, c2, c3 = nf1, nf2, nf3, nc1, nc2, nc3
    f_ref[0] = f1
    f_ref[1] = f2
    f_ref[2] = f3
    c_ref[0] = c1
    c_ref[1] = c2
    c_ref[2] = c3

    @pl.when(i == NBLK - 1)
    def _final():
        ff1, ff2, ff3 = f1, f2, f3
        cc1, cc2, cc3 = c1, c2, c3
        outs = []
        outi = []
        for t in range(KTOP):
            m = jnp.max(ff1, axis=1, keepdims=True)
            c = jnp.min(jnp.where(ff1 == m, cc1, jnp.int32(2**30)),
                        axis=1, keepdims=True)
            outs.append(m)
            outi.append(c)
            if t < KTOP - 1:
                hit = cc1 == c
                ff1 = jnp.where(hit, ff2, ff1)
                cc1 = jnp.where(hit, cc2, cc1)
                ff2 = jnp.where(hit, ff3, ff2)
                cc2 = jnp.where(hit, cc3, cc2)
                ff3 = jnp.where(hit, NEG, ff3)
        ts_ref[...] = jnp.concatenate(outs, axis=1)
        ti_ref[...] = jnp.concatenate(outi, axis=1)


def _search(queries, db_weight, interpret=False):
    return pl.pallas_call(
        _search_body,
        grid=(NBLK,),
        in_specs=[
            pl.BlockSpec((Q, D), lambda i: (0, 0)),
            pl.BlockSpec((D, BN), lambda i: (0, i)),
        ],
        out_specs=[
            pl.BlockSpec((BN, D), lambda i: (i, 0)),
            pl.BlockSpec((Q, KTOP), lambda i: (0, 0)),
            pl.BlockSpec((Q, KTOP), lambda i: (0, 0)),
        ],
        out_shape=[
            jax.ShapeDtypeStruct((NBLK * BN, D), jnp.float32),
            jax.ShapeDtypeStruct((Q, KTOP), jnp.float32),
            jax.ShapeDtypeStruct((Q, KTOP), jnp.int32),
        ],
        scratch_shapes=[
            pltpu.VMEM((Q, D), jnp.float32),
            pltpu.VMEM((KTOP, Q, 128), jnp.float32),
            pltpu.VMEM((KTOP, Q, 128), jnp.int32),
        ],
        compiler_params=pltpu.CompilerParams(
            dimension_semantics=("arbitrary",)),
        interpret=interpret,
    )(queries, db_weight)


def _gather(db_t, db_label, idx_flat):
    info = plsc.get_sparse_core_info()
    nw = info.num_cores * info.num_subcores
    b = Q * KTOP
    bpw = b // nw
    mesh = plsc.VectorSubcoreMesh(core_axis_name="c", subcore_axis_name="s")

    @functools.partial(
        pl.kernel, mesh=mesh,
        out_type=[jax.ShapeDtypeStruct((b, D), jnp.float32),
                  jax.ShapeDtypeStruct((b,), jnp.int32)],
        scratch_types=[pltpu.VMEM((bpw,), jnp.int32),
                       pltpu.VMEM((bpw, D), jnp.float32),
                       pltpu.VMEM((bpw,), jnp.int32),
                       pltpu.SemaphoreType.DMA],
    )
    def gk(table_hbm, lbl_hbm, idx_hbm, seq_out, lbl_out, idx_v, rows_v, lv, sem):
        wid = lax.axis_index("s") * info.num_cores + lax.axis_index("c")
        base = wid * bpw
        pltpu.sync_copy(idx_hbm.at[pl.ds(base, bpw)], idx_v)
        pltpu.async_copy(table_hbm.at[idx_v], rows_v, sem).wait()
        pltpu.sync_copy(rows_v, seq_out.at[pl.ds(base, bpw)])
        pltpu.async_copy(lbl_hbm.at[idx_v], lv, sem).wait()
        pltpu.sync_copy(lv, lbl_out.at[pl.ds(base, bpw)])

    return gk(db_t, db_label, idx_flat)


def _mea_body(seq_ref, q_ref, sc_ref, lb_ref, wqkv_ref, wo_ref,
              bqkv_ref, bo_ref, dw_ref, dbias_ref, ow_ref,
              ob_ref, o_ref):
    r = KTOP * RQ
    qb = q_ref[...]
    sc = sc_ref[...]
    lb = lb_ref[...]
    diota = lax.broadcasted_iota(jnp.int32, (KTOP, RQ, D), 2)
    cls3 = jnp.where(lb == diota, 1.0, 0.0) * sc
    h0 = cls3.reshape(r, D)
    h1 = jnp.concatenate([qb, qb, qb], axis=0)
    h2 = seq_ref[...].reshape(r, D)
    scale = 1.0 / np.sqrt(D // 8)
    x = jnp.concatenate([h0, h1, h2], axis=0)
    for i in range(NLAYER):
        last = i == NLAYER - 1
        bo = bo_ref[i][None, :]
        xo = (_bdot(x, wqkv_ref[i]) + bqkv_ref[i][None, :])
        qs = [xo[t * r:(t + 1) * r, 0:D] for t in range(3)]
        ks = [_round_bf16(xo[t * r:(t + 1) * r, D:2 * D]) for t in range(3)]
        vs = [_round_bf16(xo[t * r:(t + 1) * r, 2 * D:3 * D])
              for t in range(3)]
        hn = []
        for ti in range(1 if last else 3):
            qr = _round_bf16(qs[ti])
            a = [jnp.sum(qr * ks[tj], axis=1, keepdims=True) * scale
                 for tj in range(3)]
            m = jnp.maximum(jnp.maximum(a[0], a[1]), a[2])
            e = [jnp.exp(v - m) for v in a]
            den = e[0] + e[1] + e[2]
            w = [_round_bf16(v / den) for v in e]
            attn = w[0] * vs[0] + w[1] * vs[1] + w[2] * vs[2]
            hn.append(attn)
        att_x = hn[0] if last else jnp.concatenate(hn, axis=0)
        x = _bdot(att_x, wo_ref[i]) + bo
    x = jnp.tanh(_bdot(x, dw_ref[...]) + dbias_ref[...][None, :])
    lg = _bdot(x, ow_ref[...]) + ob_ref[...][None, :]
    mea = jnp.mean(lg.reshape(KTOP, RQ, NLAB), axis=0)
    liota = lax.broadcasted_iota(jnp.int32, (KTOP, RQ, NLAB), 2)
    agg = jnp.sum(jnp.where(lb == liota, 1.0, 0.0), axis=0)
    ret = agg / jnp.sum(agg, axis=1, keepdims=True)
    o_ref[...] = mea * (1.0 - RATIO) + ret * RATIO


def _mea(seqs3, queries, ts_j, lb_j, WQ, WK, WV, WO, bQ, bK, bV, bO,
         dense_w, dense_b, out_w, out_b, interpret=False):
    nblk = Q // RQ
    wqkv = jnp.concatenate([WQ, WK, WV], axis=2)
    bqkv = jnp.concatenate([bQ, bK, bV], axis=1)
    return pl.pallas_call(
        _mea_body,
        grid=(nblk,),
        in_specs=[
            pl.BlockSpec((KTOP, RQ, D), lambda s: (0, s, 0)),
            pl.BlockSpec((RQ, D), lambda s: (s, 0)),
            pl.BlockSpec((KTOP, RQ, 1), lambda s: (0, s, 0)),
            pl.BlockSpec((KTOP, RQ, 1), lambda s: (0, s, 0)),
            pl.BlockSpec((NLAYER, D, 3 * D), lambda s: (0, 0, 0)),
            pl.BlockSpec((NLAYER, D, D), lambda s: (0, 0, 0)),
            pl.BlockSpec((NLAYER, 3 * D), lambda s: (0, 0)),
            pl.BlockSpec((NLAYER, D), lambda s: (0, 0)),
            pl.BlockSpec((D, D), lambda s: (0, 0)),
            pl.BlockSpec((D,), lambda s: (0,)),
            pl.BlockSpec((D, NLAB), lambda s: (0, 0)),
            pl.BlockSpec((NLAB,), lambda s: (0,)),
        ],
        out_specs=pl.BlockSpec((RQ, NLAB), lambda s: (s, 0)),
        out_shape=jax.ShapeDtypeStruct((Q, NLAB), jnp.float32),
        compiler_params=pltpu.CompilerParams(
            dimension_semantics=("arbitrary",)),
        interpret=interpret,
    )(seqs3, queries, ts_j, lb_j, wqkv, WO, bqkv, bO,
      dense_w, dense_b, out_w, out_b)


def kernel(queries, db_weight, db_label, WQ, WK, WV, WO, bQ, bK, bV, bO,
           dense_w, dense_b, out_w, out_b):
    db_t, ts, ti = _search(queries, db_weight)
    idx_jm = ti.T.reshape(-1)
    seqs, lbls = _gather(db_t, db_label, idx_jm)
    seqs3 = seqs.reshape(KTOP, Q, D)
    ts_j = ts.T[:, :, None]
    lb_j = lbls.reshape(KTOP, Q)[:, :, None]
    return _mea(seqs3, queries, ts_j, lb_j, WQ, WK, WV, WO, bQ, bK, bV, bO,
                dense_w, dense_b, out_w, out_b)


# speedup vs baseline: 1.1601x; 1.1601x over previous
import functools

import jax
import jax.numpy as jnp
import numpy as np
from jax import lax
from jax.experimental import pallas as pl
from jax.experimental.pallas import tpu as pltpu
from jax.experimental.pallas import tpu_sc as plsc

D = 640
N = 100000
Q = 1024
KTOP = 3
NLAB = 12
NLAYER = 3
RATIO = 0.2

BN = 2048
NBLK = (N + BN - 1) // BN
RQ = 128
NEG = float("-inf")


def _bdot(a, b):
    return jnp.dot(a, b, preferred_element_type=jnp.float32)


def _round_bf16(a):
    return a.astype(jnp.bfloat16).astype(jnp.float32)


def _search_body(q_ref, db_ref, dbt_ref, ts_ref, ti_ref, qn_ref, f_ref, c_ref):
    i = pl.program_id(0)

    @pl.when(i == 0)
    def _init():
        q = q_ref[...]
        nrm = jnp.sqrt(jnp.sum(q * q, axis=1, keepdims=True))
        qn_ref[...] = q / nrm
        f_ref[...] = jnp.full((KTOP, Q, 128), NEG, jnp.float32)
        c_ref[...] = jnp.zeros((KTOP, Q, 128), jnp.int32)

    db = db_ref[...]
    dbt_ref[...] = db.T
    qn = qn_ref[...]
    lane = lax.broadcasted_iota(jnp.int32, (Q, 128), 1)
    base = i * BN
    f1 = f_ref[0]
    f2 = f_ref[1]
    f3 = f_ref[2]
    c1 = c_ref[0]
    c2 = c_ref[1]
    c3 = c_ref[2]
    s = jnp.dot(qn, db, preferred_element_type=jnp.float32)
    for g in range(BN // 128):
        v = s[:, g * 128:(g + 1) * 128]
        gb = base + g * 128
        v = jnp.where(lane < (N - gb), v, NEG)
        cc = lane + gb
        gt1 = v > f1
        gt2 = v > f2
        gt3 = v > f3
        nf3 = jnp.where(gt2, f2, jnp.where(gt3, v, f3))
        nc3 = jnp.where(gt2, c2, jnp.where(gt3, cc, c3))
        nf2 = jnp.where(gt1, f1, jnp.where(gt2, v, f2))
        nc2 = jnp.where(gt1, c1, jnp.where(gt2, cc, c2))
        nf1 = jnp.where(gt1, v, f1)
        nc1 = jnp.where(gt1, cc, c1)
        f1, f2, f3, c1, c2, c3 = nf1, nf2, nf3, nc1, nc2, nc3
    f_ref[0] = f1
    f_ref[1] = f2
    f_ref[2] = f3
    c_ref[0] = c1
    c_ref[1] = c2
    c_ref[2] = c3

    @pl.when(i == NBLK - 1)
    def _final():
        ff1, ff2, ff3 = f1, f2, f3
        cc1, cc2, cc3 = c1, c2, c3
        outs = []
        outi = []
        for t in range(KTOP):
            m = jnp.max(ff1, axis=1, keepdims=True)
            c = jnp.min(jnp.where(ff1 == m, cc1, jnp.int32(2**30)),
                        axis=1, keepdims=True)
            outs.append(m)
            outi.append(c)
            if t < KTOP - 1:
                hit = cc1 == c
                ff1 = jnp.where(hit, ff2, ff1)
                cc1 = jnp.where(hit, cc2, cc1)
                ff2 = jnp.where(hit, ff3, ff2)
                cc2 = jnp.where(hit, cc3, cc2)
                ff3 = jnp.where(hit, NEG, ff3)
        ts_ref[...] = jnp.concatenate(outs, axis=1)
        ti_ref[...] = jnp.concatenate(outi, axis=1)


def _search(queries, db_weight, interpret=False):
    return pl.pallas_call(
        _search_body,
        grid=(NBLK,),
        in_specs=[
            pl.BlockSpec((Q, D), lambda i: (0, 0)),
            pl.BlockSpec((D, BN), lambda i: (0, i)),
        ],
        out_specs=[
            pl.BlockSpec((BN, D), lambda i: (i, 0)),
            pl.BlockSpec((Q, KTOP), lambda i: (0, 0)),
            pl.BlockSpec((Q, KTOP), lambda i: (0, 0)),
        ],
        out_shape=[
            jax.ShapeDtypeStruct((NBLK * BN, D), jnp.float32),
            jax.ShapeDtypeStruct((Q, KTOP), jnp.float32),
            jax.ShapeDtypeStruct((Q, KTOP), jnp.int32),
        ],
        scratch_shapes=[
            pltpu.VMEM((Q, D), jnp.float32),
            pltpu.VMEM((KTOP, Q, 128), jnp.float32),
            pltpu.VMEM((KTOP, Q, 128), jnp.int32),
        ],
        compiler_params=pltpu.CompilerParams(
            dimension_semantics=("arbitrary",)),
        interpret=interpret,
    )(queries, db_weight)


def _gather(db_t, db_label, idx_flat):
    info = plsc.get_sparse_core_info()
    nw = info.num_cores * info.num_subcores
    b = Q * KTOP
    bpw = b // nw
    mesh = plsc.VectorSubcoreMesh(core_axis_name="c", subcore_axis_name="s")

    @functools.partial(
        pl.kernel, mesh=mesh,
        out_type=[jax.ShapeDtypeStruct((b, D), jnp.float32),
                  jax.ShapeDtypeStruct((b,), jnp.int32)],
        scratch_types=[pltpu.VMEM((bpw,), jnp.int32),
                       pltpu.VMEM((bpw, D), jnp.float32),
                       pltpu.VMEM((bpw,), jnp.int32),
                       pltpu.SemaphoreType.DMA],
    )
    def gk(table_hbm, lbl_hbm, idx_hbm, seq_out, lbl_out, idx_v, rows_v, lv, sem):
        wid = lax.axis_index("s") * info.num_cores + lax.axis_index("c")
        base = wid * bpw
        pltpu.sync_copy(idx_hbm.at[pl.ds(base, bpw)], idx_v)
        pltpu.async_copy(table_hbm.at[idx_v], rows_v, sem).wait()
        pltpu.sync_copy(rows_v, seq_out.at[pl.ds(base, bpw)])
        pltpu.async_copy(lbl_hbm.at[idx_v], lv, sem).wait()
        pltpu.sync_copy(lv, lbl_out.at[pl.ds(base, bpw)])

    return gk(db_t, db_label, idx_flat)


def _mea_body(seq_ref, q_ref, sc_ref, lb_ref, wqkv_ref, wo_ref,
              bqkv_ref, bo_ref, dw_ref, dbias_ref, ow_ref,
              ob_ref, o_ref):
    r = KTOP * RQ
    qb = q_ref[...]
    sc = sc_ref[...]
    lb = lb_ref[...]
    diota = lax.broadcasted_iota(jnp.int32, (KTOP, RQ, D), 2)
    cls3 = jnp.where(lb == diota, 1.0, 0.0) * sc
    h0 = cls3.reshape(r, D)
    h1 = jnp.concatenate([qb, qb, qb], axis=0)
    h2 = seq_ref[...].reshape(r, D)
    scale = 1.0 / np.sqrt(D // 8)
    x = jnp.concatenate([h0, h1, h2], axis=0)
    for i in range(NLAYER):
        last = i == NLAYER - 1
        bo = bo_ref[i][None, :]
        wqkv = wqkv_ref[i]
        xs = [x[t * r:(t + 1) * r, :] for t in range(3)]
        qs = [_bdot(xs[t], wqkv[:, 0:D]) + bqkv_ref[i][None, 0:D]
              for t in range(1 if last else 3)]
        ks = [_round_bf16(_bdot(xs[t], wqkv[:, D:2 * D])
                          + bqkv_ref[i][None, D:2 * D]) for t in range(3)]
        vs = [_round_bf16(_bdot(xs[t], wqkv[:, 2 * D:3 * D])
                          + bqkv_ref[i][None, 2 * D:3 * D]) for t in range(3)]
        hn = []
        for ti in range(1 if last else 3):
            qr = _round_bf16(qs[ti])
            a = [jnp.sum(qr * ks[tj], axis=1, keepdims=True) * scale
                 for tj in range(3)]
            m = jnp.maximum(jnp.maximum(a[0], a[1]), a[2])
            e = [jnp.exp(v - m) for v in a]
            den = e[0] + e[1] + e[2]
            w = [_round_bf16(v / den) for v in e]
            attn = w[0] * vs[0] + w[1] * vs[1] + w[2] * vs[2]
            hn.append(attn)
        att_x = hn[0] if last else jnp.concatenate(hn, axis=0)
        x = _bdot(att_x, wo_ref[i]) + bo
    x = jnp.tanh(_bdot(x, dw_ref[...]) + dbias_ref[...][None, :])
    lg = _bdot(x, ow_ref[...]) + ob_ref[...][None, :]
    mea = jnp.mean(lg.reshape(KTOP, RQ, NLAB), axis=0)
    liota = lax.broadcasted_iota(jnp.int32, (KTOP, RQ, NLAB), 2)
    agg = jnp.sum(jnp.where(lb == liota, 1.0, 0.0), axis=0)
    ret = agg / jnp.sum(agg, axis=1, keepdims=True)
    o_ref[...] = mea * (1.0 - RATIO) + ret * RATIO


def _mea(seqs3, queries, ts_j, lb_j, WQ, WK, WV, WO, bQ, bK, bV, bO,
         dense_w, dense_b, out_w, out_b, interpret=False):
    nblk = Q // RQ
    wqkv = jnp.concatenate([WQ, WK, WV], axis=2)
    bqkv = jnp.concatenate([bQ, bK, bV], axis=1)
    return pl.pallas_call(
        _mea_body,
        grid=(nblk,),
        in_specs=[
            pl.BlockSpec((KTOP, RQ, D), lambda s: (0, s, 0)),
            pl.BlockSpec((RQ, D), lambda s: (s, 0)),
            pl.BlockSpec((KTOP, RQ, 1), lambda s: (0, s, 0)),
            pl.BlockSpec((KTOP, RQ, 1), lambda s: (0, s, 0)),
            pl.BlockSpec((NLAYER, D, 3 * D), lambda s: (0, 0, 0)),
            pl.BlockSpec((NLAYER, D, D), lambda s: (0, 0, 0)),
            pl.BlockSpec((NLAYER, 3 * D), lambda s: (0, 0)),
            pl.BlockSpec((NLAYER, D), lambda s: (0, 0)),
            pl.BlockSpec((D, D), lambda s: (0, 0)),
            pl.BlockSpec((D,), lambda s: (0,)),
            pl.BlockSpec((D, NLAB), lambda s: (0, 0)),
            pl.BlockSpec((NLAB,), lambda s: (0,)),
        ],
        out_specs=pl.BlockSpec((RQ, NLAB), lambda s: (s, 0)),
        out_shape=jax.ShapeDtypeStruct((Q, NLAB), jnp.float32),
        compiler_params=pltpu.CompilerParams(
            dimension_semantics=("arbitrary",)),
        interpret=interpret,
    )(seqs3, queries, ts_j, lb_j, wqkv, WO, bqkv, bO,
      dense_w, dense_b, out_w, out_b)


def kernel(queries, db_weight, db_label, WQ, WK, WV, WO, bQ, bK, bV, bO,
           dense_w, dense_b, out_w, out_b):
    db_t, ts, ti = _search(queries, db_weight)
    idx_jm = ti.T.reshape(-1)
    seqs, lbls = _gather(db_t, db_label, idx_jm)
    seqs3 = seqs.reshape(KTOP, Q, D)
    ts_j = ts.T[:, :, None]
    lb_j = lbls.reshape(KTOP, Q)[:, :, None]
    return _mea(seqs3, queries, ts_j, lb_j, WQ, WK, WV, WO, bQ, bK, bV, bO,
                dense_w, dense_b, out_w, out_b)


# speedup vs baseline: 1.1768x; 1.0144x over previous
import functools

import jax
import jax.numpy as jnp
import numpy as np
from jax import lax
from jax.experimental import pallas as pl
from jax.experimental.pallas import tpu as pltpu
from jax.experimental.pallas import tpu_sc as plsc

D = 640
N = 100000
Q = 1024
KTOP = 3
NLAB = 12
NLAYER = 3
RATIO = 0.2

BN = 2048
NBLK = (N + BN - 1) // BN
RQ = 128
NEG = float("-inf")


def _bdot(a, b):
    return jnp.dot(a, b, preferred_element_type=jnp.float32)


def _round_bf16(a):
    return a.astype(jnp.bfloat16).astype(jnp.float32)


def _search_body(q_ref, db_ref, dbt_ref, ts_ref, ti_ref, qn_ref, f_ref, c_ref):
    i = pl.program_id(0)

    @pl.when(i == 0)
    def _init():
        q = q_ref[...]
        nrm = jnp.sqrt(jnp.sum(q * q, axis=1, keepdims=True))
        qn_ref[...] = q / nrm
        f_ref[...] = jnp.full((KTOP, Q, 128), NEG, jnp.float32)
        c_ref[...] = jnp.zeros((KTOP, Q, 128), jnp.int32)

    db = db_ref[...]
    dbt_ref[...] = db.T
    qn = qn_ref[...]
    lane = lax.broadcasted_iota(jnp.int32, (Q, 128), 1)
    base = i * BN
    f1 = f_ref[0]
    f2 = f_ref[1]
    f3 = f_ref[2]
    c1 = c_ref[0]
    c2 = c_ref[1]
    c3 = c_ref[2]
    s = jnp.dot(qn, db, preferred_element_type=jnp.float32)

    f1 = f_ref[0]
    f2 = f_ref[1]
    f3 = f_ref[2]
    c1 = c_ref[0]
    c2 = c_ref[1]
    c3 = c_ref[2]
    for g in range(BN // 128):
        v = s[:, g * 128:(g + 1) * 128]
        gb = base + g * 128
        v = jnp.where(lane < (N - gb), v, NEG)
        cc = lane + gb
        gt1 = v > f1
        gt2 = v > f2
        gt3 = v > f3
        nf3 = jnp.where(gt2, f2, jnp.where(gt3, v, f3))
        nc3 = jnp.where(gt2, c2, jnp.where(gt3, cc, c3))
        nf2 = jnp.where(gt1, f1, jnp.where(gt2, v, f2))
        nc2 = jnp.where(gt1, c1, jnp.where(gt2, cc, c2))
        nf1 = jnp.where(gt1, v, f1)
        nc1 = jnp.where(gt1, cc, c1)
        f1, f2, f3, c1, c2, c3 = nf1, nf2, nf3, nc1, nc2, nc3
    f_ref[0] = f1
    f_ref[1] = f2
    f_ref[2] = f3
    c_ref[0] = c1
    c_ref[1] = c2
    c_ref[2] = c3

    @pl.when(i == NBLK - 1)
    def _last():
        ff1, ff2, ff3 = f1, f2, f3
        cc1, cc2, cc3 = c1, c2, c3
        for t in range(KTOP):
            m = jnp.max(ff1, axis=1, keepdims=True)
            c = jnp.min(jnp.where(ff1 == m, cc1, jnp.int32(2**30)),
                        axis=1, keepdims=True)
            ts_ref[:, t:t + 1] = m
            ti_ref[:, t:t + 1] = c
            if t < KTOP - 1:
                hit = cc1 == c
                ff1 = jnp.where(hit, ff2, ff1)
                cc1 = jnp.where(hit, cc2, cc1)
                ff2 = jnp.where(hit, ff3, ff2)
                cc2 = jnp.where(hit, cc3, cc2)
                ff3 = jnp.where(hit, NEG, ff3)


def _search(queries, db_weight, interpret=False):
    return pl.pallas_call(
        _search_body,
        grid=(NBLK,),
        in_specs=[
            pl.BlockSpec((Q, D), lambda i: (0, 0)),
            pl.BlockSpec((D, BN), lambda i: (0, i)),
        ],
        out_specs=[
            pl.BlockSpec((BN, D), lambda i: (i, 0)),
            pl.BlockSpec((Q, KTOP), lambda i: (0, 0)),
            pl.BlockSpec((Q, KTOP), lambda i: (0, 0)),
        ],
        out_shape=[
            jax.ShapeDtypeStruct((NBLK * BN, D), jnp.float32),
            jax.ShapeDtypeStruct((Q, KTOP), jnp.float32),
            jax.ShapeDtypeStruct((Q, KTOP), jnp.int32),
        ],
        scratch_shapes=[
            pltpu.VMEM((Q, D), jnp.float32),
            pltpu.VMEM((KTOP, Q, 128), jnp.float32),
            pltpu.VMEM((KTOP, Q, 128), jnp.int32),
        ],
        compiler_params=pltpu.CompilerParams(
            dimension_semantics=("arbitrary",)),
        interpret=interpret,
    )(queries, db_weight)


def _gather(db_t, db_label, idx_flat):
    info = plsc.get_sparse_core_info()
    nw = info.num_cores * info.num_subcores
    b = Q * KTOP
    bpw = b // nw
    mesh = plsc.VectorSubcoreMesh(core_axis_name="c", subcore_axis_name="s")

    @functools.partial(
        pl.kernel, mesh=mesh,
        out_type=[jax.ShapeDtypeStruct((b, D), jnp.float32),
                  jax.ShapeDtypeStruct((b,), jnp.int32)],
        scratch_types=[pltpu.VMEM((bpw,), jnp.int32),
                       pltpu.VMEM((bpw, D), jnp.float32),
                       pltpu.VMEM((bpw,), jnp.int32),
                       pltpu.SemaphoreType.DMA],
    )
    def gk(table_hbm, lbl_hbm, idx_hbm, seq_out, lbl_out, idx_v, rows_v, lv, sem):
        wid = lax.axis_index("s") * info.num_cores + lax.axis_index("c")
        base = wid * bpw
        pltpu.sync_copy(idx_hbm.at[pl.ds(base, bpw)], idx_v)
        pltpu.async_copy(table_hbm.at[idx_v], rows_v, sem).wait()
        pltpu.sync_copy(rows_v, seq_out.at[pl.ds(base, bpw)])
        pltpu.async_copy(lbl_hbm.at[idx_v], lv, sem).wait()
        pltpu.sync_copy(lv, lbl_out.at[pl.ds(base, bpw)])

    return gk(db_t, db_label, idx_flat)


def _mea_body(seq_ref, q_ref, sc_ref, lb_ref, wq_ref, wk_ref, wv_ref, wo_ref,
              bq_ref, bk_ref, bv_ref, bo_ref, dw_ref, dbias_ref, ow_ref,
              ob_ref, o_ref):
    r = KTOP * RQ
    qb = q_ref[...]
    sc = sc_ref[...]
    lb = lb_ref[...]
    diota = lax.broadcasted_iota(jnp.int32, (KTOP, RQ, D), 2)
    cls3 = jnp.where(lb == diota, 1.0, 0.0) * sc
    h0 = cls3.reshape(r, D)
    h1 = jnp.concatenate([qb, qb, qb], axis=0)
    h2 = seq_ref[...].reshape(r, D)
    scale = 1.0 / np.sqrt(D // 8)
    x = jnp.concatenate([h0, h1, h2], axis=0)
    for i in range(NLAYER):
        last = i == NLAYER - 1
        bo = bo_ref[i][None, :]
        xs = [x[t * r:(t + 1) * r, :] for t in range(3)]
        qs = [_bdot(xs[t], wq_ref[i]) + bq_ref[i][None, :]
              for t in range(1 if last else 3)]
        ks = [_round_bf16(_bdot(xs[t], wk_ref[i]) + bk_ref[i][None, :])
              for t in range(3)]
        vs = [_round_bf16(_bdot(xs[t], wv_ref[i]) + bv_ref[i][None, :])
              for t in range(3)]
        hn = []
        for ti in range(1 if last else 3):
            qr = _round_bf16(qs[ti])
            a = [jnp.sum(qr * ks[tj], axis=1, keepdims=True) * scale
                 for tj in range(3)]
            m = jnp.maximum(jnp.maximum(a[0], a[1]), a[2])
            e = [jnp.exp(v - m) for v in a]
            den = e[0] + e[1] + e[2]
            w = [_round_bf16(v / den) for v in e]
            attn = w[0] * vs[0] + w[1] * vs[1] + w[2] * vs[2]
            hn.append(attn)
        att_x = hn[0] if last else jnp.concatenate(hn, axis=0)
        x = _bdot(att_x, wo_ref[i]) + bo
    x = jnp.tanh(_bdot(x, dw_ref[...]) + dbias_ref[...][None, :])
    lg = _bdot(x, ow_ref[...]) + ob_ref[...][None, :]
    mea = jnp.mean(lg.reshape(KTOP, RQ, NLAB), axis=0)
    liota = lax.broadcasted_iota(jnp.int32, (KTOP, RQ, NLAB), 2)
    agg = jnp.sum(jnp.where(lb == liota, 1.0, 0.0), axis=0)
    ret = agg / jnp.sum(agg, axis=1, keepdims=True)
    o_ref[...] = mea * (1.0 - RATIO) + ret * RATIO


def _mea(seqs3, queries, ts_j, lb_j, WQ, WK, WV, WO, bQ, bK, bV, bO,
         dense_w, dense_b, out_w, out_b, interpret=False):
    nblk = Q // RQ
    return pl.pallas_call(
        _mea_body,
        grid=(nblk,),
        in_specs=[
            pl.BlockSpec((KTOP, RQ, D), lambda s: (0, s, 0)),
            pl.BlockSpec((RQ, D), lambda s: (s, 0)),
            pl.BlockSpec((KTOP, RQ, 1), lambda s: (0, s, 0)),
            pl.BlockSpec((KTOP, RQ, 1), lambda s: (0, s, 0)),
            pl.BlockSpec((NLAYER, D, D), lambda s: (0, 0, 0)),
            pl.BlockSpec((NLAYER, D, D), lambda s: (0, 0, 0)),
            pl.BlockSpec((NLAYER, D, D), lambda s: (0, 0, 0)),
            pl.BlockSpec((NLAYER, D, D), lambda s: (0, 0, 0)),
            pl.BlockSpec((NLAYER, D), lambda s: (0, 0)),
            pl.BlockSpec((NLAYER, D), lambda s: (0, 0)),
            pl.BlockSpec((NLAYER, D), lambda s: (0, 0)),
            pl.BlockSpec((NLAYER, D), lambda s: (0, 0)),
            pl.BlockSpec((D, D), lambda s: (0, 0)),
            pl.BlockSpec((D,), lambda s: (0,)),
            pl.BlockSpec((D, NLAB), lambda s: (0, 0)),
            pl.BlockSpec((NLAB,), lambda s: (0,)),
        ],
        out_specs=pl.BlockSpec((RQ, NLAB), lambda s: (s, 0)),
        out_shape=jax.ShapeDtypeStruct((Q, NLAB), jnp.float32),
        compiler_params=pltpu.CompilerParams(
            dimension_semantics=("arbitrary",)),
        interpret=interpret,
    )(seqs3, queries, ts_j, lb_j, WQ, WK, WV, WO, bQ, bK, bV, bO,
      dense_w, dense_b, out_w, out_b)


def kernel(queries, db_weight, db_label, WQ, WK, WV, WO, bQ, bK, bV, bO,
           dense_w, dense_b, out_w, out_b):
    db_t, ts, ti = _search(queries, db_weight)
    idx_jm = ti.T.reshape(-1)
    seqs, lbls = _gather(db_t, db_label, idx_jm)
    seqs3 = seqs.reshape(KTOP, Q, D)
    ts_j = ts.T[:, :, None]
    lb_j = lbls.reshape(KTOP, Q)[:, :, None]
    return _mea(seqs3, queries, ts_j, lb_j, WQ, WK, WV, WO, bQ, bK, bV, bO,
                dense_w, dense_b, out_w, out_b)


# speedup vs baseline: 1.6920x; 1.4378x over previous
import functools

import jax
import jax.numpy as jnp
import numpy as np
from jax import lax
from jax.experimental import pallas as pl
from jax.experimental.pallas import tpu as pltpu
from jax.experimental.pallas import tpu_sc as plsc

D = 640
N = 100000
Q = 1024
KTOP = 3
NLAB = 12
NLAYER = 3
RATIO = 0.2

BN = 2048
NBLK = (N + BN - 1) // BN
RQ = 128
NEG = float("-inf")


def _bdot(a, b):
    return jnp.dot(a, b, preferred_element_type=jnp.float32)


def _round_bf16(a):
    return a.astype(jnp.bfloat16).astype(jnp.float32)


def _search_body(q_ref, db_ref, ts_ref, ti_ref, qn_ref, f_ref, c_ref):
    i = pl.program_id(0)

    @pl.when(i == 0)
    def _init():
        q = q_ref[...]
        nrm = jnp.sqrt(jnp.sum(q * q, axis=1, keepdims=True))
        qn_ref[...] = q / nrm
        f_ref[...] = jnp.full((KTOP, Q, 128), NEG, jnp.float32)
        c_ref[...] = jnp.zeros((KTOP, Q, 128), jnp.int32)

    db = db_ref[...]
    qn = qn_ref[...]
    lane = lax.broadcasted_iota(jnp.int32, (Q, 128), 1)
    base = i * BN
    s = lax.dot_general(qn, db, (((1,), (1,)), ((), ())),
                        preferred_element_type=jnp.float32)

    f1 = f_ref[0]
    f2 = f_ref[1]
    f3 = f_ref[2]
    c1 = c_ref[0]
    c2 = c_ref[1]
    c3 = c_ref[2]
    for g in range(BN // 128):
        v = s[:, g * 128:(g + 1) * 128]
        gb = base + g * 128
        v = jnp.where(lane < (N - gb), v, NEG)
        cc = lane + gb
        gt1 = v > f1
        gt2 = v > f2
        gt3 = v > f3
        nf3 = jnp.where(gt2, f2, jnp.where(gt3, v, f3))
        nc3 = jnp.where(gt2, c2, jnp.where(gt3, cc, c3))
        nf2 = jnp.where(gt1, f1, jnp.where(gt2, v, f2))
        nc2 = jnp.where(gt1, c1, jnp.where(gt2, cc, c2))
        nf1 = jnp.where(gt1, v, f1)
        nc1 = jnp.where(gt1, cc, c1)
        f1, f2, f3, c1, c2, c3 = nf1, nf2, nf3, nc1, nc2, nc3
    f_ref[0] = f1
    f_ref[1] = f2
    f_ref[2] = f3
    c_ref[0] = c1
    c_ref[1] = c2
    c_ref[2] = c3

    @pl.when(i == NBLK - 1)
    def _last():
        ff1, ff2, ff3 = f1, f2, f3
        cc1, cc2, cc3 = c1, c2, c3
        for t in range(KTOP):
            m = jnp.max(ff1, axis=1, keepdims=True)
            c = jnp.min(jnp.where(ff1 == m, cc1, jnp.int32(2**30)),
                        axis=1, keepdims=True)
            ts_ref[:, t:t + 1] = m
            ti_ref[:, t:t + 1] = c
            if t < KTOP - 1:
                hit = cc1 == c
                ff1 = jnp.where(hit, ff2, ff1)
                cc1 = jnp.where(hit, cc2, cc1)
                ff2 = jnp.where(hit, ff3, ff2)
                cc2 = jnp.where(hit, cc3, cc2)
                ff3 = jnp.where(hit, NEG, ff3)


def _search(queries, db_wt, interpret=False):
    return pl.pallas_call(
        _search_body,
        grid=(NBLK,),
        in_specs=[
            pl.BlockSpec((Q, D), lambda i: (0, 0)),
            pl.BlockSpec((BN, D), lambda i: (i, 0)),
        ],
        out_specs=[
            pl.BlockSpec((Q, KTOP), lambda i: (0, 0)),
            pl.BlockSpec((Q, KTOP), lambda i: (0, 0)),
        ],
        out_shape=[
            jax.ShapeDtypeStruct((Q, KTOP), jnp.float32),
            jax.ShapeDtypeStruct((Q, KTOP), jnp.int32),
        ],
        scratch_shapes=[
            pltpu.VMEM((Q, D), jnp.float32),
            pltpu.VMEM((KTOP, Q, 128), jnp.float32),
            pltpu.VMEM((KTOP, Q, 128), jnp.int32),
        ],
        compiler_params=pltpu.CompilerParams(
            dimension_semantics=("arbitrary",)),
        interpret=interpret,
    )(queries, db_wt)


def _gather(db_t, db_label, idx_flat):
    info = plsc.get_sparse_core_info()
    nw = info.num_cores * info.num_subcores
    b = Q * KTOP
    bpw = b // nw
    mesh = plsc.VectorSubcoreMesh(core_axis_name="c", subcore_axis_name="s")

    @functools.partial(
        pl.kernel, mesh=mesh,
        out_type=[jax.ShapeDtypeStruct((b, D), jnp.float32),
                  jax.ShapeDtypeStruct((b,), jnp.int32)],
        scratch_types=[pltpu.VMEM((bpw,), jnp.int32),
                       pltpu.VMEM((bpw, D), jnp.float32),
                       pltpu.VMEM((bpw,), jnp.int32),
                       pltpu.SemaphoreType.DMA],
    )
    def gk(table_hbm, lbl_hbm, idx_hbm, seq_out, lbl_out, idx_v, rows_v, lv, sem):
        wid = lax.axis_index("s") * info.num_cores + lax.axis_index("c")
        base = wid * bpw
        pltpu.sync_copy(idx_hbm.at[pl.ds(base, bpw)], idx_v)
        pltpu.async_copy(table_hbm.at[idx_v], rows_v, sem).wait()
        pltpu.sync_copy(rows_v, seq_out.at[pl.ds(base, bpw)])
        pltpu.async_copy(lbl_hbm.at[idx_v], lv, sem).wait()
        pltpu.sync_copy(lv, lbl_out.at[pl.ds(base, bpw)])

    return gk(db_t, db_label, idx_flat)


def _mea_body(seq_ref, q_ref, sc_ref, lb_ref, wq_ref, wk_ref, wv_ref, wo_ref,
              bq_ref, bk_ref, bv_ref, bo_ref, dw_ref, dbias_ref, ow_ref,
              ob_ref, o_ref):
    r = KTOP * RQ
    qb = q_ref[...]
    sc = sc_ref[...]
    lb = lb_ref[...]
    diota = lax.broadcasted_iota(jnp.int32, (KTOP, RQ, D), 2)
    cls3 = jnp.where(lb == diota, 1.0, 0.0) * sc
    h0 = cls3.reshape(r, D)
    h1 = jnp.concatenate([qb, qb, qb], axis=0)
    h2 = seq_ref[...].reshape(r, D)
    scale = 1.0 / np.sqrt(D // 8)
    x = jnp.concatenate([h0, h1, h2], axis=0)
    for i in range(NLAYER):
        last = i == NLAYER - 1
        bo = bo_ref[i][None, :]
        xs = [x[t * r:(t + 1) * r, :] for t in range(3)]
        qs = [_bdot(xs[t], wq_ref[i]) + bq_ref[i][None, :]
              for t in range(1 if last else 3)]
        ks = [_round_bf16(_bdot(xs[t], wk_ref[i]) + bk_ref[i][None, :])
              for t in range(3)]
        vs = [_round_bf16(_bdot(xs[t], wv_ref[i]) + bv_ref[i][None, :])
              for t in range(3)]
        hn = []
        for ti in range(1 if last else 3):
            qr = _round_bf16(qs[ti])
            a = [jnp.sum(qr * ks[tj], axis=1, keepdims=True) * scale
                 for tj in range(3)]
            m = jnp.maximum(jnp.maximum(a[0], a[1]), a[2])
            e = [jnp.exp(v - m) for v in a]
            den = e[0] + e[1] + e[2]
            w = [_round_bf16(v / den) for v in e]
            attn = w[0] * vs[0] + w[1] * vs[1] + w[2] * vs[2]
            hn.append(attn)
        att_x = hn[0] if last else jnp.concatenate(hn, axis=0)
        x = _bdot(att_x, wo_ref[i]) + bo
    x = jnp.tanh(_bdot(x, dw_ref[...]) + dbias_ref[...][None, :])
    lg = _bdot(x, ow_ref[...]) + ob_ref[...][None, :]
    mea = jnp.mean(lg.reshape(KTOP, RQ, NLAB), axis=0)
    liota = lax.broadcasted_iota(jnp.int32, (KTOP, RQ, NLAB), 2)
    agg = jnp.sum(jnp.where(lb == liota, 1.0, 0.0), axis=0)
    ret = agg / jnp.sum(agg, axis=1, keepdims=True)
    o_ref[...] = mea * (1.0 - RATIO) + ret * RATIO


def _mea(seqs3, queries, ts_j, lb_j, WQ, WK, WV, WO, bQ, bK, bV, bO,
         dense_w, dense_b, out_w, out_b, interpret=False):
    nblk = Q // RQ
    return pl.pallas_call(
        _mea_body,
        grid=(nblk,),
        in_specs=[
            pl.BlockSpec((KTOP, RQ, D), lambda s: (0, s, 0)),
            pl.BlockSpec((RQ, D), lambda s: (s, 0)),
            pl.BlockSpec((KTOP, RQ, 1), lambda s: (0, s, 0)),
            pl.BlockSpec((KTOP, RQ, 1), lambda s: (0, s, 0)),
            pl.BlockSpec((NLAYER, D, D), lambda s: (0, 0, 0)),
            pl.BlockSpec((NLAYER, D, D), lambda s: (0, 0, 0)),
            pl.BlockSpec((NLAYER, D, D), lambda s: (0, 0, 0)),
            pl.BlockSpec((NLAYER, D, D), lambda s: (0, 0, 0)),
            pl.BlockSpec((NLAYER, D), lambda s: (0, 0)),
            pl.BlockSpec((NLAYER, D), lambda s: (0, 0)),
            pl.BlockSpec((NLAYER, D), lambda s: (0, 0)),
            pl.BlockSpec((NLAYER, D), lambda s: (0, 0)),
            pl.BlockSpec((D, D), lambda s: (0, 0)),
            pl.BlockSpec((D,), lambda s: (0,)),
            pl.BlockSpec((D, NLAB), lambda s: (0, 0)),
            pl.BlockSpec((NLAB,), lambda s: (0,)),
        ],
        out_specs=pl.BlockSpec((RQ, NLAB), lambda s: (s, 0)),
        out_shape=jax.ShapeDtypeStruct((Q, NLAB), jnp.float32),
        compiler_params=pltpu.CompilerParams(
            dimension_semantics=("arbitrary",)),
        interpret=interpret,
    )(seqs3, queries, ts_j, lb_j, WQ, WK, WV, WO, bQ, bK, bV, bO,
      dense_w, dense_b, out_w, out_b)


def kernel(queries, db_weight, db_label, WQ, WK, WV, WO, bQ, bK, bV, bO,
           dense_w, dense_b, out_w, out_b):
    db_wt = db_weight.T
    ts, ti = _search(queries, db_wt)
    idx_jm = ti.T.reshape(-1)
    seqs, lbls = _gather(db_wt, db_label, idx_jm)
    seqs3 = seqs.reshape(KTOP, Q, D)
    ts_j = ts.T[:, :, None]
    lb_j = lbls.reshape(KTOP, Q)[:, :, None]
    return _mea(seqs3, queries, ts_j, lb_j, WQ, WK, WV, WO, bQ, bK, bV, bO,
                dense_w, dense_b, out_w, out_b)
